# Initial kernel scaffold; baseline (speedup 1.0000x reference)
import jax
import jax.numpy as jnp
from jax.experimental import pallas as pl


def kernel(label, fixed, emb_table, final_w, final_b):
    raise NotImplementedError("write your pallas kernel here")


# speedup vs baseline: 1.3202x; 1.3202x over previous
import functools

import jax
import jax.numpy as jnp
from jax import lax
from jax.experimental import pallas as pl
from jax.experimental.pallas import tpu as pltpu
from jax.experimental.pallas import tpu_sc as plsc

H = 16
LANES = 16
NC, NS = 2, 16
NW = NC * NS


def _sc_embed_sq(label_flat, emb_table, B, L):
    bpw = B // NW
    GP = LANES
    n_groups = bpw // GP
    ROWS = GP * L
    CH = 80
    NCH = ROWS // CH

    mesh = plsc.VectorSubcoreMesh(core_axis_name="c", subcore_axis_name="s")

    @functools.partial(
        pl.kernel,
        out_type=jax.ShapeDtypeStruct((B * H,), jnp.float32),
        mesh=mesh,
        scratch_types=[
            pltpu.VMEM((bpw * L,), jnp.int32),
            pltpu.VMEM((ROWS, H), jnp.float32),
            pltpu.VMEM((GP * H,), jnp.float32),
            pltpu.SemaphoreType.DMA,
        ],
        compiler_params=pltpu.CompilerParams(
            needs_layout_passes=False, use_tc_tiling_on_sc=False),
    )
    def k(label_hbm, table_hbm, out_hbm, lab_v, rows_v, outb_v, sem):
        wid = lax.axis_index("s") * NC + lax.axis_index("c")
        base = wid * bpw
        pltpu.sync_copy(label_hbm.at[pl.ds(base * L, bpw * L)], lab_v)
        lane_iota = lax.iota(jnp.int32, LANES)
        row_iota = lane_iota * L

        def per_group(g, carry):
            goff = pl.multiple_of(g * ROWS, 8)
            cps = [
                pltpu.async_copy(
                    table_hbm.at[lab_v.at[pl.ds(goff + j * CH, CH)]],
                    rows_v.at[pl.ds(j * CH, CH)],
                    sem,
                )
                for j in range(NCH)
            ]
            for cp in cps:
                cp.wait()

            def per_step(l, accs):
                ridx = row_iota + l
                cols = [
                    plsc.load_gather(
                        rows_v, [ridx, jnp.full((LANES,), c, jnp.int32)])
                    for c in range(H)
                ]
                s = cols[0] * cols[0]
                for c in range(1, H):
                    s = s + cols[c] * cols[c]
                scale2 = jnp.where(s > 1.0, 1.0 / s, 1.0)
                new = []
                for c in range(H):
                    p = jnp.maximum(cols[c], 0.0)
                    new.append(accs[c] + p * p * scale2)
                return tuple(new)

            accs = lax.fori_loop(
                0, L, per_step,
                tuple(jnp.zeros((LANES,), jnp.float32) for _ in range(H)))
            out_iota = lane_iota * H
            for c in range(H):
                plsc.store_scatter(outb_v, [out_iota + c], accs[c])
            pltpu.sync_copy(
                outb_v, out_hbm.at[pl.ds((base + g * GP) * H, GP * H)])
            return carry

        lax.fori_loop(0, n_groups, per_group, 0)

    return k(label_flat, emb_table)


def _tc_finalize(acc, fixed, wmT, we, wfT, bias):
    B = acc.shape[0]
    F = fixed.shape[1]
    BLK = 4096
    grid = (B // BLK,)

    def body(acc_ref, fixed_ref, wmT_ref, we_ref, wfT_ref, b_ref, out_ref):
        embed = jnp.sqrt(acc_ref[...])
        fx = fixed_ref[...]
        v = jnp.dot(fx, wmT_ref[...], preferred_element_type=jnp.float32)
        v = v + we_ref[...]
        s1 = jnp.dot(fx, wfT_ref[...], preferred_element_type=jnp.float32)
        logit = jnp.sum(embed * v, axis=1, keepdims=True) + s1 + b_ref[...]
        out_ref[...] = jax.nn.sigmoid(logit)

    return pl.pallas_call(
        body,
        grid=grid,
        in_specs=[
            pl.BlockSpec((BLK, H), lambda i: (i, 0)),
            pl.BlockSpec((BLK, F), lambda i: (i, 0)),
            pl.BlockSpec((F, H), lambda i: (0, 0)),
            pl.BlockSpec((1, H), lambda i: (0, 0)),
            pl.BlockSpec((F, 1), lambda i: (0, 0)),
            pl.BlockSpec((1, 1), lambda i: (0, 0)),
        ],
        out_specs=pl.BlockSpec((BLK, 1), lambda i: (i, 0)),
        out_shape=jax.ShapeDtypeStruct((B, 1), jnp.float32),
    )(acc, fixed, wmT, we, wfT, bias)


def kernel(label, fixed, emb_table, final_w, final_b):
    B, L = label.shape
    F = fixed.shape[1]
    acc = _sc_embed_sq(label.reshape(-1).astype(jnp.int32), emb_table, B, L)
    acc = acc.reshape(B, H)
    wfT = final_w[0, :F].reshape(F, 1)
    we = final_w[:, F:F + H]
    wmT = final_w[0, F + H:].reshape(H, F).T
    return _tc_finalize(acc, fixed, wmT, we, wfT, final_b.reshape(1, 1))


# speedup vs baseline: 1.5189x; 1.1505x over previous
import functools

import jax
import jax.numpy as jnp
from jax import lax
from jax.experimental import pallas as pl
from jax.experimental.pallas import tpu as pltpu
from jax.experimental.pallas import tpu_sc as plsc

H = 16
NC, NS = 2, 16
NW = NC * NS


def _tc_prep(tableT):
    V = tableT.shape[1]
    W = 8192
    grid = (pl.cdiv(V, W),)

    def body(t_ref, o_ref):
        e = t_ref[...]
        s = jnp.sum(e * e, axis=0, keepdims=True)
        scale2 = jnp.where(s > 1.0, 1.0 / s, 1.0)
        p = jnp.maximum(e, 0.0)
        o_ref[...] = (p * p * scale2).T

    return pl.pallas_call(
        body,
        grid=grid,
        in_specs=[pl.BlockSpec((H, W), lambda i: (0, i))],
        out_specs=pl.BlockSpec((W, H), lambda i: (i, 0)),
        out_shape=jax.ShapeDtypeStruct((V, H), jnp.float32),
    )(tableT)


def _sc_gather_sum(label_flat, contrib, B, L):
    bpw = B // NW
    GP = 32
    n_groups = bpw // GP
    ROWS = GP * L
    CH = 128
    NCH = ROWS // CH
    REM = ROWS - NCH * CH
    UNROLL = 10

    mesh = plsc.VectorSubcoreMesh(core_axis_name="c", subcore_axis_name="s")

    @functools.partial(
        pl.kernel,
        out_type=jax.ShapeDtypeStruct((B * H,), jnp.float32),
        mesh=mesh,
        scratch_types=[
            pltpu.VMEM((bpw * L,), jnp.int32),
            pltpu.VMEM((ROWS, H), jnp.float32),
            pltpu.VMEM((ROWS, H), jnp.float32),
            pltpu.VMEM((GP * H,), jnp.float32),
            pltpu.SemaphoreType.DMA,
            pltpu.SemaphoreType.DMA,
        ],
        compiler_params=pltpu.CompilerParams(
            needs_layout_passes=False, use_tc_tiling_on_sc=False),
    )
    def k(label_hbm, table_hbm, out_hbm, lab_v, rows0, rows1, outb_v,
          sem0, sem1):
        wid = lax.axis_index("s") * NC + lax.axis_index("c")
        base = wid * bpw
        pltpu.sync_copy(label_hbm.at[pl.ds(base * L, bpw * L)], lab_v)

        def chunks(g, rows_v, sem):
            goff = pl.multiple_of(g * ROWS, 8)
            cps = []
            for j in range(NCH):
                cps.append(pltpu.make_async_copy(
                    table_hbm.at[lab_v.at[pl.ds(goff + j * CH, CH)]],
                    rows_v.at[pl.ds(j * CH, CH)], sem))
            cps.append(pltpu.make_async_copy(
                table_hbm.at[lab_v.at[pl.ds(goff + NCH * CH, REM)]],
                rows_v.at[pl.ds(NCH * CH, REM)], sem))
            return cps

        def fire(g, rows_v, sem):
            for cp in chunks(g, rows_v, sem):
                cp.start()

        def drain(g, rows_v, sem):
            for cp in chunks(g, rows_v, sem):
                cp.wait()

        def consume(g, rows_v):
            def per_b(b, carry):
                def accum(i, acc):
                    r = b * L + i * UNROLL
                    for u in range(UNROLL):
                        acc = acc + rows_v[r + u, :]
                    return acc
                acc = lax.fori_loop(
                    0, L // UNROLL, accum, jnp.zeros((H,), jnp.float32))
                outb_v[pl.ds(pl.multiple_of(b * H, 16), H)] = acc
                return carry
            lax.fori_loop(0, GP, per_b, 0)
            pltpu.sync_copy(
                outb_v, out_hbm.at[pl.ds((base + g * GP) * H, GP * H)])

        fire(0, rows0, sem0)

        def per_pair(i, carry):
            g0 = i * 2
            fire(g0 + 1, rows1, sem1)
            drain(g0, rows0, sem0)
            consume(g0, rows0)

            @pl.when(g0 + 2 < n_groups)
            def _():
                fire(g0 + 2, rows0, sem0)

            drain(g0 + 1, rows1, sem1)
            consume(g0 + 1, rows1)
            return carry

        lax.fori_loop(0, n_groups // 2, per_pair, 0)

    return k(label_flat, contrib)


def _tc_finalize(acc, fixed, wmT, we, wfT, bias):
    B = acc.shape[0]
    F = fixed.shape[1]
    BLK = 4096
    grid = (B // BLK,)

    def body(acc_ref, fixed_ref, wmT_ref, we_ref, wfT_ref, b_ref, out_ref):
        embed = jnp.sqrt(acc_ref[...])
        fx = fixed_ref[...]
        v = jnp.dot(fx, wmT_ref[...], preferred_element_type=jnp.float32)
        v = v + we_ref[...]
        s1 = jnp.dot(fx, wfT_ref[...], preferred_element_type=jnp.float32)
        logit = jnp.sum(embed * v, axis=1, keepdims=True) + s1 + b_ref[...]
        out_ref[...] = jax.nn.sigmoid(logit)

    return pl.pallas_call(
        body,
        grid=grid,
        in_specs=[
            pl.BlockSpec((BLK, H), lambda i: (i, 0)),
            pl.BlockSpec((BLK, F), lambda i: (i, 0)),
            pl.BlockSpec((F, H), lambda i: (0, 0)),
            pl.BlockSpec((1, H), lambda i: (0, 0)),
            pl.BlockSpec((F, 1), lambda i: (0, 0)),
            pl.BlockSpec((1, 1), lambda i: (0, 0)),
        ],
        out_specs=pl.BlockSpec((BLK, 1), lambda i: (i, 0)),
        out_shape=jax.ShapeDtypeStruct((B, 1), jnp.float32),
    )(acc, fixed, wmT, we, wfT, bias)


def kernel(label, fixed, emb_table, final_w, final_b):
    B, L = label.shape
    F = fixed.shape[1]
    contrib = _tc_prep(emb_table.T)
    acc = _sc_gather_sum(label.reshape(-1).astype(jnp.int32), contrib, B, L)
    acc = acc.reshape(B, H)
    wfT = final_w[0, :F].reshape(F, 1)
    we = final_w[:, F:F + H]
    wmT = final_w[0, F + H:].reshape(H, F).T
    return _tc_finalize(acc, fixed, wmT, we, wfT, final_b.reshape(1, 1))


# speedup vs baseline: 2.5212x; 1.6599x over previous
import functools

import jax
import jax.numpy as jnp
from jax import lax
from jax.experimental import pallas as pl
from jax.experimental.pallas import tpu as pltpu
from jax.experimental.pallas import tpu_sc as plsc

H = 16
NC, NS = 2, 16
NW = NC * NS


PREP_W = 8192
PREP_J = PREP_W // 8


def _tc_prep(tableT):
    V = tableT.shape[1]
    NB = pl.cdiv(V, PREP_W)
    grid = (NB,)

    def body(t_ref, o_ref):
        e = t_ref[...]
        s = jnp.sum(e * e, axis=0, keepdims=True)
        scale2 = jnp.where(s > 1.0, 1.0 / s, 1.0)
        p = jnp.maximum(e, 0.0)
        contrib = (p * p * scale2).T
        o_ref[...] = jnp.concatenate(
            [contrib[PREP_J * t:PREP_J * (t + 1), :] for t in range(8)],
            axis=1)

    return pl.pallas_call(
        body,
        grid=grid,
        in_specs=[pl.BlockSpec((H, PREP_W), lambda i: (0, i))],
        out_specs=pl.BlockSpec((PREP_J, 128), lambda i: (i, 0)),
        out_shape=jax.ShapeDtypeStruct((NB * PREP_J, 128), jnp.float32),
    )(tableT)


def _sc_gather_sum(label_flat, contrib, B, L):
    bpw = B // NW
    GP = 32
    n_groups = bpw // GP
    ROWS = GP * L
    CH = 128
    NCH = ROWS // CH
    REM = ROWS - NCH * CH
    UNROLL = 10

    mesh = plsc.VectorSubcoreMesh(core_axis_name="c", subcore_axis_name="s")

    @functools.partial(
        pl.kernel,
        out_type=jax.ShapeDtypeStruct((B * H,), jnp.float32),
        mesh=mesh,
        scratch_types=[
            pltpu.VMEM((bpw * L,), jnp.int32),
            pltpu.VMEM((ROWS, H), jnp.float32),
            pltpu.VMEM((ROWS, H), jnp.float32),
            pltpu.VMEM((GP * H,), jnp.float32),
            pltpu.SemaphoreType.DMA,
            pltpu.SemaphoreType.DMA,
        ],
        compiler_params=pltpu.CompilerParams(
            needs_layout_passes=False, use_tc_tiling_on_sc=False),
    )
    def k(label_hbm, table_hbm, out_hbm, lab_v, rows0, rows1, outb_v,
          sem0, sem1):
        wid = lax.axis_index("s") * NC + lax.axis_index("c")
        base = wid * bpw
        pltpu.sync_copy(label_hbm.at[pl.ds(base * L, bpw * L)], lab_v)

        def remap(i, carry):
            off = pl.multiple_of(i * H, 16)
            r = lab_v[pl.ds(off, H)]
            w = jnp.bitwise_and(r, PREP_W - 1)
            p = ((r - w) + jnp.left_shift(jnp.bitwise_and(w, PREP_J - 1), 3)
                 + jnp.right_shift(w, 10))
            lab_v[pl.ds(off, H)] = p
            return carry

        lax.fori_loop(0, bpw * L // H, remap, 0)

        def chunks(g, rows_v, sem):
            goff = pl.multiple_of(g * ROWS, 8)
            cps = []
            for j in range(NCH):
                cps.append(pltpu.make_async_copy(
                    table_hbm.at[lab_v.at[pl.ds(goff + j * CH, CH)]],
                    rows_v.at[pl.ds(j * CH, CH)], sem))
            cps.append(pltpu.make_async_copy(
                table_hbm.at[lab_v.at[pl.ds(goff + NCH * CH, REM)]],
                rows_v.at[pl.ds(NCH * CH, REM)], sem))
            return cps

        def fire(g, rows_v, sem):
            for cp in chunks(g, rows_v, sem):
                cp.start()

        def drain(g, rows_v, sem):
            for cp in chunks(g, rows_v, sem):
                cp.wait()

        def consume(g, rows_v):
            def per_b(b, carry):
                def accum(i, acc):
                    r = b * L + i * UNROLL
                    for u in range(UNROLL):
                        acc = acc + rows_v[r + u, :]
                    return acc
                acc = lax.fori_loop(
                    0, L // UNROLL, accum, jnp.zeros((H,), jnp.float32))
                outb_v[pl.ds(pl.multiple_of(b * H, 16), H)] = acc
                return carry
            lax.fori_loop(0, GP, per_b, 0)
            pltpu.sync_copy(
                outb_v, out_hbm.at[pl.ds((base + g * GP) * H, GP * H)])

        fire(0, rows0, sem0)

        def per_pair(i, carry):
            g0 = i * 2
            fire(g0 + 1, rows1, sem1)
            drain(g0, rows0, sem0)
            consume(g0, rows0)

            @pl.when(g0 + 2 < n_groups)
            def _():
                fire(g0 + 2, rows0, sem0)

            drain(g0 + 1, rows1, sem1)
            consume(g0 + 1, rows1)
            return carry

        lax.fori_loop(0, n_groups // 2, per_pair, 0)

    return k(label_flat, contrib)


def _tc_finalize(acc, fixed, wmT, we, wfT, bias):
    B = acc.shape[0]
    F = fixed.shape[1]
    BLK = 4096
    grid = (B // BLK,)

    def body(acc_ref, fixed_ref, wmT_ref, we_ref, wfT_ref, b_ref, out_ref):
        embed = jnp.sqrt(acc_ref[...])
        fx = fixed_ref[...]
        v = jnp.dot(fx, wmT_ref[...], preferred_element_type=jnp.float32)
        v = v + we_ref[...]
        s1 = jnp.dot(fx, wfT_ref[...], preferred_element_type=jnp.float32)
        logit = jnp.sum(embed * v, axis=1, keepdims=True) + s1 + b_ref[...]
        out_ref[...] = jax.nn.sigmoid(logit)

    return pl.pallas_call(
        body,
        grid=grid,
        in_specs=[
            pl.BlockSpec((BLK, H), lambda i: (i, 0)),
            pl.BlockSpec((BLK, F), lambda i: (i, 0)),
            pl.BlockSpec((F, H), lambda i: (0, 0)),
            pl.BlockSpec((1, H), lambda i: (0, 0)),
            pl.BlockSpec((F, 1), lambda i: (0, 0)),
            pl.BlockSpec((1, 1), lambda i: (0, 0)),
        ],
        out_specs=pl.BlockSpec((BLK, 1), lambda i: (i, 0)),
        out_shape=jax.ShapeDtypeStruct((B, 1), jnp.float32),
    )(acc, fixed, wmT, we, wfT, bias)


def kernel(label, fixed, emb_table, final_w, final_b):
    B, L = label.shape
    F = fixed.shape[1]
    packed = _tc_prep(emb_table.T)
    contrib = packed.reshape(packed.shape[0] * 8, H)
    acc = _sc_gather_sum(label.reshape(-1).astype(jnp.int32), contrib, B, L)
    acc = acc.reshape(B, H)
    wfT = final_w[0, :F].reshape(F, 1)
    we = final_w[:, F:F + H]
    wmT = final_w[0, F + H:].reshape(H, F).T
    return _tc_finalize(acc, fixed, wmT, we, wfT, final_b.reshape(1, 1))


# speedup vs baseline: 4.5729x; 1.8138x over previous
import functools

import jax
import jax.numpy as jnp
from jax import lax
from jax.experimental import pallas as pl
from jax.experimental.pallas import tpu as pltpu
from jax.experimental.pallas import tpu_sc as plsc

H = 16
NC, NS = 2, 16
NW = NC * NS


PREP_W = 8192
PREP_J = PREP_W // 8


def _tc_prep(tableT):
    V = tableT.shape[1]
    NB = pl.cdiv(V, PREP_W)
    grid = (NB,)

    def body(t_ref, o_ref):
        e = t_ref[...]
        s = jnp.sum(e * e, axis=0, keepdims=True)
        scale2 = jnp.where(s > 1.0, 1.0 / s, 1.0)
        p = jnp.maximum(e, 0.0)
        contrib = p * p * scale2
        for q in range(8):
            x = jnp.concatenate(
                [contrib[:, 128 * (8 * q + u):128 * (8 * q + u + 1)]
                 for u in range(8)], axis=0)
            o_ref[128 * q:128 * (q + 1), :] = x.T

    return pl.pallas_call(
        body,
        grid=grid,
        in_specs=[pl.BlockSpec((H, PREP_W), lambda i: (0, i))],
        out_specs=pl.BlockSpec((PREP_J, 128), lambda i: (i, 0)),
        out_shape=jax.ShapeDtypeStruct((NB * PREP_J, 128), jnp.float32),
        compiler_params=pltpu.CompilerParams(
            fuse_transposed_lhs_in_matmul=True),
    )(tableT)


def _sc_gather_sum(label_flat, contrib, B, L):
    bpw = B // NW
    GP = 32
    n_groups = bpw // GP
    ROWS = GP * L
    CH = 128
    NCH = ROWS // CH
    REM = ROWS - NCH * CH
    UNROLL = 10

    mesh = plsc.VectorSubcoreMesh(core_axis_name="c", subcore_axis_name="s")

    @functools.partial(
        pl.kernel,
        out_type=jax.ShapeDtypeStruct((B * H,), jnp.float32),
        mesh=mesh,
        scratch_types=[
            pltpu.VMEM((bpw * L,), jnp.int32),
            pltpu.VMEM((ROWS, H), jnp.float32),
            pltpu.VMEM((ROWS, H), jnp.float32),
            pltpu.VMEM((GP * H,), jnp.float32),
            pltpu.SemaphoreType.DMA,
            pltpu.SemaphoreType.DMA,
        ],
        compiler_params=pltpu.CompilerParams(
            needs_layout_passes=False, use_tc_tiling_on_sc=False),
    )
    def k(label_hbm, table_hbm, out_hbm, lab_v, rows0, rows1, outb_v,
          sem0, sem1):
        wid = lax.axis_index("s") * NC + lax.axis_index("c")
        base = wid * bpw
        pltpu.sync_copy(label_hbm.at[pl.ds(base * L, bpw * L)], lab_v)

        def remap(i, carry):
            off = pl.multiple_of(i * H, 16)
            r = lab_v[pl.ds(off, H)]
            w = jnp.bitwise_and(r, PREP_W - 1)
            p = ((r - w) + jnp.bitwise_and(w, 7168)
                 + jnp.left_shift(jnp.bitwise_and(w, 127), 3)
                 + jnp.bitwise_and(jnp.right_shift(w, 7), 7))
            lab_v[pl.ds(off, H)] = p
            return carry

        lax.fori_loop(0, bpw * L // H, remap, 0)

        def chunks(g, rows_v, sem):
            goff = pl.multiple_of(g * ROWS, 8)
            cps = []
            for j in range(NCH):
                cps.append(pltpu.make_async_copy(
                    table_hbm.at[lab_v.at[pl.ds(goff + j * CH, CH)]],
                    rows_v.at[pl.ds(j * CH, CH)], sem))
            cps.append(pltpu.make_async_copy(
                table_hbm.at[lab_v.at[pl.ds(goff + NCH * CH, REM)]],
                rows_v.at[pl.ds(NCH * CH, REM)], sem))
            return cps

        def fire(g, rows_v, sem):
            for cp in chunks(g, rows_v, sem):
                cp.start()

        def drain(g, rows_v, sem):
            for cp in chunks(g, rows_v, sem):
                cp.wait()

        def consume(g, rows_v):
            def per_b(b, carry):
                def accum(i, acc):
                    r = b * L + i * UNROLL
                    for u in range(UNROLL):
                        acc = acc + rows_v[r + u, :]
                    return acc
                acc = lax.fori_loop(
                    0, L // UNROLL, accum, jnp.zeros((H,), jnp.float32))
                outb_v[pl.ds(pl.multiple_of(b * H, 16), H)] = acc
                return carry
            lax.fori_loop(0, GP, per_b, 0)
            pltpu.sync_copy(
                outb_v, out_hbm.at[pl.ds((base + g * GP) * H, GP * H)])

        fire(0, rows0, sem0)

        def per_pair(i, carry):
            g0 = i * 2
            fire(g0 + 1, rows1, sem1)
            drain(g0, rows0, sem0)
            consume(g0, rows0)

            @pl.when(g0 + 2 < n_groups)
            def _():
                fire(g0 + 2, rows0, sem0)

            drain(g0 + 1, rows1, sem1)
            consume(g0 + 1, rows1)
            return carry

        lax.fori_loop(0, n_groups // 2, per_pair, 0)

    return k(label_flat, contrib)


def _tc_finalize(acc, fixed, wmT, we, wfT, bias):
    B = acc.shape[0]
    F = fixed.shape[1]
    BLK = 4096
    grid = (B // BLK,)

    def body(acc_ref, fixed_ref, wmT_ref, we_ref, wfT_ref, b_ref, out_ref):
        embed = jnp.sqrt(acc_ref[...])
        fx = fixed_ref[...]
        v = jnp.dot(fx, wmT_ref[...], preferred_element_type=jnp.float32)
        v = v + we_ref[...]
        s1 = jnp.dot(fx, wfT_ref[...], preferred_element_type=jnp.float32)
        logit = jnp.sum(embed * v, axis=1, keepdims=True) + s1 + b_ref[...]
        out_ref[...] = jax.nn.sigmoid(logit)

    return pl.pallas_call(
        body,
        grid=grid,
        in_specs=[
            pl.BlockSpec((BLK, H), lambda i: (i, 0)),
            pl.BlockSpec((BLK, F), lambda i: (i, 0)),
            pl.BlockSpec((F, H), lambda i: (0, 0)),
            pl.BlockSpec((1, H), lambda i: (0, 0)),
            pl.BlockSpec((F, 1), lambda i: (0, 0)),
            pl.BlockSpec((1, 1), lambda i: (0, 0)),
        ],
        out_specs=pl.BlockSpec((BLK, 1), lambda i: (i, 0)),
        out_shape=jax.ShapeDtypeStruct((B, 1), jnp.float32),
    )(acc, fixed, wmT, we, wfT, bias)


def kernel(label, fixed, emb_table, final_w, final_b):
    B, L = label.shape
    F = fixed.shape[1]
    packed = _tc_prep(emb_table.T)
    contrib = packed.reshape(packed.shape[0] * 8, H)
    acc = _sc_gather_sum(label.reshape(-1).astype(jnp.int32), contrib, B, L)
    acc = acc.reshape(B, H)
    wfT = final_w[0, :F].reshape(F, 1)
    we = final_w[:, F:F + H]
    wmT = final_w[0, F + H:].reshape(H, F).T
    return _tc_finalize(acc, fixed, wmT, we, wfT, final_b.reshape(1, 1))


# speedup vs baseline: 4.9392x; 1.0801x over previous
import functools

import jax
import jax.numpy as jnp
from jax import lax
from jax.experimental import pallas as pl
from jax.experimental.pallas import tpu as pltpu
from jax.experimental.pallas import tpu_sc as plsc

H = 16
NC, NS = 2, 16
NW = NC * NS


PREP_W = 8192
PREP_J = PREP_W // 8


def _tc_prep(tableT):
    V = tableT.shape[1]
    NB = pl.cdiv(V, PREP_W)
    grid = (NB,)

    def body(t_ref, o_ref):
        e = t_ref[...]
        s = jnp.sum(e * e, axis=0, keepdims=True)
        scale2 = jnp.where(s > 1.0, 1.0 / s, 1.0)
        p = jnp.maximum(e, 0.0)
        contrib = p * p * scale2
        for q in range(8):
            x = jnp.concatenate(
                [contrib[:, 128 * (8 * q + u):128 * (8 * q + u + 1)]
                 for u in range(8)], axis=0)
            o_ref[128 * q:128 * (q + 1), :] = x.T

    return pl.pallas_call(
        body,
        grid=grid,
        in_specs=[pl.BlockSpec((H, PREP_W), lambda i: (0, i))],
        out_specs=pl.BlockSpec((PREP_J, 128), lambda i: (i, 0)),
        out_shape=jax.ShapeDtypeStruct((NB * PREP_J, 128), jnp.float32),
        compiler_params=pltpu.CompilerParams(
            fuse_transposed_lhs_in_matmul=True),
    )(tableT)


def _sc_gather_sum(label_flat, contrib, B, L):
    bpw = B // NW
    GP = 32
    n_groups = bpw // GP
    ROWS = GP * L
    CH = 128
    NCH = ROWS // CH
    REM = ROWS - NCH * CH
    UNROLL = 10

    mesh = plsc.VectorSubcoreMesh(core_axis_name="c", subcore_axis_name="s")

    @functools.partial(
        pl.kernel,
        out_type=jax.ShapeDtypeStruct((H * B,), jnp.float32),
        mesh=mesh,
        scratch_types=[
            pltpu.VMEM((bpw * L,), jnp.int32),
            pltpu.VMEM((ROWS, H), jnp.float32),
            pltpu.VMEM((ROWS, H), jnp.float32),
            pltpu.VMEM((H * GP,), jnp.float32),
            pltpu.SemaphoreType.DMA,
            pltpu.SemaphoreType.DMA,
        ],
        compiler_params=pltpu.CompilerParams(
            needs_layout_passes=False, use_tc_tiling_on_sc=False),
    )
    def k(label_hbm, table_hbm, out_hbm, lab_v, rows0, rows1, outb_v,
          sem0, sem1):
        wid = lax.axis_index("s") * NC + lax.axis_index("c")
        base = wid * bpw
        pltpu.sync_copy(label_hbm.at[pl.ds(base * L, bpw * L)], lab_v)

        def remap(i, carry):
            off = pl.multiple_of(i * H, 16)
            r = lab_v[pl.ds(off, H)]
            w = jnp.bitwise_and(r, PREP_W - 1)
            p = ((r - w) + jnp.bitwise_and(w, 7168)
                 + jnp.left_shift(jnp.bitwise_and(w, 127), 3)
                 + jnp.bitwise_and(jnp.right_shift(w, 7), 7))
            lab_v[pl.ds(off, H)] = p
            return carry

        lax.fori_loop(0, bpw * L // H, remap, 0)

        def chunks(g, rows_v, sem):
            goff = pl.multiple_of(g * ROWS, 8)
            cps = []
            for j in range(NCH):
                cps.append(pltpu.make_async_copy(
                    table_hbm.at[lab_v.at[pl.ds(goff + j * CH, CH)]],
                    rows_v.at[pl.ds(j * CH, CH)], sem))
            cps.append(pltpu.make_async_copy(
                table_hbm.at[lab_v.at[pl.ds(goff + NCH * CH, REM)]],
                rows_v.at[pl.ds(NCH * CH, REM)], sem))
            return cps

        def fire(g, rows_v, sem):
            for cp in chunks(g, rows_v, sem):
                cp.start()

        def drain(g, rows_v, sem):
            for cp in chunks(g, rows_v, sem):
                cp.wait()

        col_iota = lax.iota(jnp.int32, H) * GP

        def consume(g, rows_v):
            def per_b(b, carry):
                r0 = b * L
                seed = tuple(rows_v[r0 + u, :] for u in range(4))

                def accum(i, accs4):
                    r = r0 + 4 + i * 4
                    a0, a1, a2, a3 = accs4
                    return (a0 + rows_v[r, :], a1 + rows_v[r + 1, :],
                            a2 + rows_v[r + 2, :], a3 + rows_v[r + 3, :])

                n4 = (L - 4) // 4
                accs = list(lax.fori_loop(0, n4, accum, seed))
                for u in range((L - 4) % 4):
                    accs[u] = accs[u] + rows_v[r0 + 4 + n4 * 4 + u, :]
                acc = (accs[0] + accs[1]) + (accs[2] + accs[3])
                plsc.store_scatter(outb_v, [col_iota + b], acc)
                return carry
            lax.fori_loop(0, GP, per_b, 0)
            for c in range(H):
                pltpu.sync_copy(
                    outb_v.at[pl.ds(c * GP, GP)],
                    out_hbm.at[pl.ds(c * B + base + g * GP, GP)])

        fire(0, rows0, sem0)

        def per_pair(i, carry):
            g0 = i * 2
            fire(g0 + 1, rows1, sem1)
            drain(g0, rows0, sem0)
            consume(g0, rows0)

            @pl.when(g0 + 2 < n_groups)
            def _():
                fire(g0 + 2, rows0, sem0)

            drain(g0 + 1, rows1, sem1)
            consume(g0 + 1, rows1)
            return carry

        lax.fori_loop(0, n_groups // 2, per_pair, 0)

    return k(label_flat, contrib)


def _tc_finalize(accT, fixedT, wm, weT, wf, bias):
    B = accT.shape[1]
    F = fixedT.shape[0]
    BLK = 4096
    grid = (B // BLK,)

    def body(acc_ref, fx_ref, wm_ref, weT_ref, wf_ref, b_ref, out_ref):
        embed = jnp.sqrt(acc_ref[...])
        fx = fx_ref[...]
        v = jnp.dot(wm_ref[...], fx, preferred_element_type=jnp.float32)
        v = v + weT_ref[...]
        s1 = jnp.dot(wf_ref[...], fx, preferred_element_type=jnp.float32)
        logit = jnp.sum(embed * v, axis=0, keepdims=True) + s1 + b_ref[...]
        out_ref[...] = jax.nn.sigmoid(logit)

    return pl.pallas_call(
        body,
        grid=grid,
        in_specs=[
            pl.BlockSpec((H, BLK), lambda i: (0, i)),
            pl.BlockSpec((F, BLK), lambda i: (0, i)),
            pl.BlockSpec((H, F), lambda i: (0, 0)),
            pl.BlockSpec((H, 1), lambda i: (0, 0)),
            pl.BlockSpec((1, F), lambda i: (0, 0)),
            pl.BlockSpec((1, 1), lambda i: (0, 0)),
        ],
        out_specs=pl.BlockSpec((1, BLK), lambda i: (0, i)),
        out_shape=jax.ShapeDtypeStruct((1, B), jnp.float32),
    )(accT, fixedT, wm, weT, wf, bias)


def kernel(label, fixed, emb_table, final_w, final_b):
    B, L = label.shape
    F = fixed.shape[1]
    packed = _tc_prep(emb_table.T)
    contrib = packed.reshape(packed.shape[0] * 8, H)
    acc = _sc_gather_sum(label.reshape(-1).astype(jnp.int32), contrib, B, L)
    accT = acc.reshape(H, B)
    wf = final_w[:, :F]
    weT = final_w[0, F:F + H].reshape(H, 1)
    wm = final_w[0, F + H:].reshape(H, F)
    out = _tc_finalize(accT, fixed.T, wm, weT, wf, final_b.reshape(1, 1))
    return out.reshape(B, 1)


# speedup vs baseline: 6.5245x; 1.3210x over previous
import functools

import jax
import jax.numpy as jnp
from jax import lax
from jax.experimental import pallas as pl
from jax.experimental.pallas import tpu as pltpu
from jax.experimental.pallas import tpu_sc as plsc

H = 16
NC, NS = 2, 16
NW = NC * NS


PREP_W = 32768
PREP_J = PREP_W // 8


def _tc_prep(tableT):
    V = tableT.shape[1]
    NB = pl.cdiv(V, PREP_W)
    grid = (NB,)

    def body(t_ref, o_ref):
        e = t_ref[...]
        s = jnp.sum(e * e, axis=0, keepdims=True)
        scale2 = jnp.where(s > 1.0, 1.0 / s, 1.0)
        p = jnp.maximum(e, 0.0)
        contrib = p * p * scale2
        for q in range(PREP_W // 1024):
            x = jnp.concatenate(
                [contrib[:, 128 * (8 * q + u):128 * (8 * q + u + 1)]
                 for u in range(8)], axis=0)
            o_ref[128 * q:128 * (q + 1), :] = x.T

    return pl.pallas_call(
        body,
        grid=grid,
        in_specs=[pl.BlockSpec((H, PREP_W), lambda i: (0, i))],
        out_specs=pl.BlockSpec((PREP_J, 128), lambda i: (i, 0)),
        out_shape=jax.ShapeDtypeStruct((NB * PREP_J, 128), jnp.float32),
        compiler_params=pltpu.CompilerParams(
            fuse_transposed_lhs_in_matmul=True),
    )(tableT)


def _sc_gather_sum(label_flat, contrib, B, L):
    bpw = B // NW
    GP = 32
    n_groups = bpw // GP
    ROWS = GP * L
    CH = 128
    NCH = ROWS // CH
    REM = ROWS - NCH * CH
    UNROLL = 10

    mesh = plsc.VectorSubcoreMesh(core_axis_name="c", subcore_axis_name="s")

    @functools.partial(
        pl.kernel,
        out_type=jax.ShapeDtypeStruct((H * B,), jnp.float32),
        mesh=mesh,
        scratch_types=[
            pltpu.VMEM((bpw * L,), jnp.int32),
            pltpu.VMEM((ROWS, H), jnp.float32),
            pltpu.VMEM((ROWS, H), jnp.float32),
            pltpu.VMEM((H * GP,), jnp.float32),
            pltpu.SemaphoreType.DMA,
            pltpu.SemaphoreType.DMA,
        ],
        compiler_params=pltpu.CompilerParams(
            needs_layout_passes=False, use_tc_tiling_on_sc=False),
    )
    def k(label_hbm, table_hbm, out_hbm, lab_v, rows0, rows1, outb_v,
          sem0, sem1):
        wid = lax.axis_index("s") * NC + lax.axis_index("c")
        base = wid * bpw
        pltpu.sync_copy(label_hbm.at[pl.ds(base * L, bpw * L)], lab_v)

        def remap(i, carry):
            off = pl.multiple_of(i * H, 16)
            r = lab_v[pl.ds(off, H)]
            p = (jnp.bitwise_and(r, -1024)
                 + jnp.left_shift(jnp.bitwise_and(r, 127), 3)
                 + jnp.bitwise_and(jnp.right_shift(r, 7), 7))
            lab_v[pl.ds(off, H)] = p
            return carry

        lax.fori_loop(0, bpw * L // H, remap, 0)

        def chunks(g, rows_v, sem):
            goff = pl.multiple_of(g * ROWS, 8)
            cps = []
            for j in range(NCH):
                cps.append(pltpu.make_async_copy(
                    table_hbm.at[lab_v.at[pl.ds(goff + j * CH, CH)]],
                    rows_v.at[pl.ds(j * CH, CH)], sem))
            cps.append(pltpu.make_async_copy(
                table_hbm.at[lab_v.at[pl.ds(goff + NCH * CH, REM)]],
                rows_v.at[pl.ds(NCH * CH, REM)], sem))
            return cps

        def fire(g, rows_v, sem):
            for cp in chunks(g, rows_v, sem):
                cp.start()

        def drain(g, rows_v, sem):
            for cp in chunks(g, rows_v, sem):
                cp.wait()

        col_iota = lax.iota(jnp.int32, H) * GP

        def consume(g, rows_v):
            def per_b(b, carry):
                r0 = b * L
                seed = tuple(rows_v[r0 + u, :] for u in range(4))

                def accum(i, accs4):
                    r = r0 + 4 + i * 4
                    a0, a1, a2, a3 = accs4
                    return (a0 + rows_v[r, :], a1 + rows_v[r + 1, :],
                            a2 + rows_v[r + 2, :], a3 + rows_v[r + 3, :])

                n4 = (L - 4) // 4
                accs = list(lax.fori_loop(0, n4, accum, seed))
                for u in range((L - 4) % 4):
                    accs[u] = accs[u] + rows_v[r0 + 4 + n4 * 4 + u, :]
                acc = (accs[0] + accs[1]) + (accs[2] + accs[3])
                plsc.store_scatter(outb_v, [col_iota + b], acc)
                return carry
            lax.fori_loop(0, GP, per_b, 0)
            for c in range(H):
                pltpu.sync_copy(
                    outb_v.at[pl.ds(c * GP, GP)],
                    out_hbm.at[pl.ds(c * B + base + g * GP, GP)])

        fire(0, rows0, sem0)

        def per_pair(i, carry):
            g0 = i * 2
            fire(g0 + 1, rows1, sem1)
            drain(g0, rows0, sem0)
            consume(g0, rows0)

            @pl.when(g0 + 2 < n_groups)
            def _():
                fire(g0 + 2, rows0, sem0)

            drain(g0 + 1, rows1, sem1)
            consume(g0 + 1, rows1)
            return carry

        lax.fori_loop(0, n_groups // 2, per_pair, 0)

    return k(label_flat, contrib)


def _tc_finalize(accT, fixedT, wm, weT, wf, bias):
    B = accT.shape[1]
    F = fixedT.shape[0]
    BLK = 4096
    grid = (B // BLK,)

    def body(acc_ref, fx_ref, wm_ref, weT_ref, wf_ref, b_ref, out_ref):
        embed = jnp.sqrt(acc_ref[...])
        fx = fx_ref[...]
        v = jnp.dot(wm_ref[...], fx, preferred_element_type=jnp.float32)
        v = v + weT_ref[...]
        s1 = jnp.dot(wf_ref[...], fx, preferred_element_type=jnp.float32)
        logit = jnp.sum(embed * v, axis=0, keepdims=True) + s1 + b_ref[...]
        out_ref[...] = jax.nn.sigmoid(logit)

    return pl.pallas_call(
        body,
        grid=grid,
        in_specs=[
            pl.BlockSpec((H, BLK), lambda i: (0, i)),
            pl.BlockSpec((F, BLK), lambda i: (0, i)),
            pl.BlockSpec((H, F), lambda i: (0, 0)),
            pl.BlockSpec((H, 1), lambda i: (0, 0)),
            pl.BlockSpec((1, F), lambda i: (0, 0)),
            pl.BlockSpec((1, 1), lambda i: (0, 0)),
        ],
        out_specs=pl.BlockSpec((1, BLK), lambda i: (0, i)),
        out_shape=jax.ShapeDtypeStruct((1, B), jnp.float32),
    )(accT, fixedT, wm, weT, wf, bias)


def kernel(label, fixed, emb_table, final_w, final_b):
    B, L = label.shape
    F = fixed.shape[1]
    packed = _tc_prep(emb_table.T)
    contrib = packed.reshape(packed.shape[0] * 8, H)
    acc = _sc_gather_sum(label.reshape(-1).astype(jnp.int32), contrib, B, L)
    accT = acc.reshape(H, B)
    wf = final_w[:, :F]
    weT = final_w[0, F:F + H].reshape(H, 1)
    wm = final_w[0, F + H:].reshape(H, F)
    out = _tc_finalize(accT, fixed.T, wm, weT, wf, final_b.reshape(1, 1))
    return out.reshape(B, 1)


# speedup vs baseline: 6.8645x; 1.0521x over previous
import functools

import jax
import jax.numpy as jnp
from jax import lax
from jax.experimental import pallas as pl
from jax.experimental.pallas import tpu as pltpu
from jax.experimental.pallas import tpu_sc as plsc

H = 16
NC, NS = 2, 16
NW = NC * NS


PREP_W = 65536
PREP_J = PREP_W // 8


def _tc_prep(tableT):
    V = tableT.shape[1]
    NB = pl.cdiv(V, PREP_W)
    grid = (NB,)

    def body(t_ref, o_ref):
        e = t_ref[...]
        s = jnp.sum(e * e, axis=0, keepdims=True)
        scale2 = jnp.where(s > 1.0, 1.0 / s, 1.0)
        p = jnp.maximum(e, 0.0)
        contrib = p * p * scale2
        for q in range(PREP_W // 1024):
            x = jnp.concatenate(
                [contrib[:, 128 * (8 * q + u):128 * (8 * q + u + 1)]
                 for u in range(8)], axis=0)
            o_ref[128 * q:128 * (q + 1), :] = x.T

    return pl.pallas_call(
        body,
        grid=grid,
        in_specs=[pl.BlockSpec((H, PREP_W), lambda i: (0, i))],
        out_specs=pl.BlockSpec((PREP_J, 128), lambda i: (i, 0)),
        out_shape=jax.ShapeDtypeStruct((NB * PREP_J, 128), jnp.float32),
        compiler_params=pltpu.CompilerParams(
            fuse_transposed_lhs_in_matmul=True),
    )(tableT)


def _sc_gather_sum(label_flat, contrib, B, L):
    bpw = B // NW
    GP = 32
    n_groups = bpw // GP
    ROWS = GP * L
    CH = 128
    NCH = ROWS // CH
    REM = ROWS - NCH * CH
    UNROLL = 10

    mesh = plsc.VectorSubcoreMesh(core_axis_name="c", subcore_axis_name="s")

    @functools.partial(
        pl.kernel,
        out_type=jax.ShapeDtypeStruct((H * B,), jnp.float32),
        mesh=mesh,
        scratch_types=[
            pltpu.VMEM((bpw * L,), jnp.int32),
            pltpu.VMEM((ROWS, H), jnp.float32),
            pltpu.VMEM((ROWS, H), jnp.float32),
            pltpu.VMEM((H * GP,), jnp.float32),
            pltpu.SemaphoreType.DMA,
            pltpu.SemaphoreType.DMA,
        ],
        compiler_params=pltpu.CompilerParams(
            needs_layout_passes=False, use_tc_tiling_on_sc=False),
    )
    def k(label_hbm, table_hbm, out_hbm, lab_v, rows0, rows1, outb_v,
          sem0, sem1):
        wid = lax.axis_index("s") * NC + lax.axis_index("c")
        base = wid * bpw
        pltpu.sync_copy(label_hbm.at[pl.ds(base * L, bpw * L)], lab_v)

        def remap(i, carry):
            off = pl.multiple_of(i * H, 16)
            r = lab_v[pl.ds(off, H)]
            p = (jnp.bitwise_and(r, -1024)
                 + jnp.left_shift(jnp.bitwise_and(r, 127), 3)
                 + jnp.bitwise_and(jnp.right_shift(r, 7), 7))
            lab_v[pl.ds(off, H)] = p
            return carry

        lax.fori_loop(0, bpw * L // H, remap, 0)

        def chunks(g, rows_v, sem):
            goff = pl.multiple_of(g * ROWS, 8)
            cps = []
            for j in range(NCH):
                cps.append(pltpu.make_async_copy(
                    table_hbm.at[lab_v.at[pl.ds(goff + j * CH, CH)]],
                    rows_v.at[pl.ds(j * CH, CH)], sem))
            cps.append(pltpu.make_async_copy(
                table_hbm.at[lab_v.at[pl.ds(goff + NCH * CH, REM)]],
                rows_v.at[pl.ds(NCH * CH, REM)], sem))
            return cps

        def fire(g, rows_v, sem):
            for cp in chunks(g, rows_v, sem):
                cp.start()

        def drain(g, rows_v, sem):
            for cp in chunks(g, rows_v, sem):
                cp.wait()

        col_iota = lax.iota(jnp.int32, H) * GP

        def consume(g, rows_v):
            def per_b(b, carry):
                r0 = b * L
                seed = tuple(rows_v[r0 + u, :] for u in range(4))

                def accum(i, accs4):
                    r = r0 + 4 + i * 4
                    a0, a1, a2, a3 = accs4
                    return (a0 + rows_v[r, :], a1 + rows_v[r + 1, :],
                            a2 + rows_v[r + 2, :], a3 + rows_v[r + 3, :])

                n4 = (L - 4) // 4
                accs = list(lax.fori_loop(0, n4, accum, seed))
                for u in range((L - 4) % 4):
                    accs[u] = accs[u] + rows_v[r0 + 4 + n4 * 4 + u, :]
                acc = (accs[0] + accs[1]) + (accs[2] + accs[3])
                plsc.store_scatter(outb_v, [col_iota + b], acc)
                return carry
            lax.fori_loop(0, GP, per_b, 0)
            for c in range(H):
                pltpu.sync_copy(
                    outb_v.at[pl.ds(c * GP, GP)],
                    out_hbm.at[pl.ds(c * B + base + g * GP, GP)])

        fire(0, rows0, sem0)

        def per_pair(i, carry):
            g0 = i * 2
            fire(g0 + 1, rows1, sem1)
            drain(g0, rows0, sem0)
            consume(g0, rows0)

            @pl.when(g0 + 2 < n_groups)
            def _():
                fire(g0 + 2, rows0, sem0)

            drain(g0 + 1, rows1, sem1)
            consume(g0 + 1, rows1)
            return carry

        lax.fori_loop(0, n_groups // 2, per_pair, 0)

    return k(label_flat, contrib)


def _tc_finalize(accT, fixedT, wm, weT, wf, bias):
    B = accT.shape[1]
    F = fixedT.shape[0]
    BLK = 4096
    grid = (B // BLK,)

    def body(acc_ref, fx_ref, wm_ref, weT_ref, wf_ref, b_ref, out_ref):
        embed = jnp.sqrt(acc_ref[...])
        fx = fx_ref[...]
        v = jnp.dot(wm_ref[...], fx, preferred_element_type=jnp.float32)
        v = v + weT_ref[...]
        s1 = jnp.dot(wf_ref[...], fx, preferred_element_type=jnp.float32)
        logit = jnp.sum(embed * v, axis=0, keepdims=True) + s1 + b_ref[...]
        out_ref[...] = jax.nn.sigmoid(logit)

    return pl.pallas_call(
        body,
        grid=grid,
        in_specs=[
            pl.BlockSpec((H, BLK), lambda i: (0, i)),
            pl.BlockSpec((F, BLK), lambda i: (0, i)),
            pl.BlockSpec((H, F), lambda i: (0, 0)),
            pl.BlockSpec((H, 1), lambda i: (0, 0)),
            pl.BlockSpec((1, F), lambda i: (0, 0)),
            pl.BlockSpec((1, 1), lambda i: (0, 0)),
        ],
        out_specs=pl.BlockSpec((1, BLK), lambda i: (0, i)),
        out_shape=jax.ShapeDtypeStruct((1, B), jnp.float32),
    )(accT, fixedT, wm, weT, wf, bias)


def kernel(label, fixed, emb_table, final_w, final_b):
    B, L = label.shape
    F = fixed.shape[1]
    packed = _tc_prep(emb_table.T)
    contrib = packed.reshape(packed.shape[0] * 8, H)
    acc = _sc_gather_sum(label.reshape(-1).astype(jnp.int32), contrib, B, L)
    accT = acc.reshape(H, B)
    wf = final_w[:, :F]
    weT = final_w[0, F:F + H].reshape(H, 1)
    wm = final_w[0, F + H:].reshape(H, F)
    out = _tc_finalize(accT, fixed.T, wm, weT, wf, final_b.reshape(1, 1))
    return out.reshape(B, 1)


# speedup vs baseline: 7.0498x; 1.0270x over previous
import functools

import jax
import jax.numpy as jnp
from jax import lax
from jax.experimental import pallas as pl
from jax.experimental.pallas import tpu as pltpu
from jax.experimental.pallas import tpu_sc as plsc

H = 16
NC, NS = 2, 16
NW = NC * NS


PREP_W = 131072
PREP_J = PREP_W // 8


def _tc_prep(tableT):
    V = tableT.shape[1]
    NB = pl.cdiv(V, PREP_W)
    grid = (NB,)

    def body(t_ref, o_ref):
        e = t_ref[...]
        s = jnp.sum(e * e, axis=0, keepdims=True)
        scale2 = jnp.where(s > 1.0, 1.0 / s, 1.0)
        p = jnp.maximum(e, 0.0)
        contrib = p * p * scale2
        for q in range(PREP_W // 1024):
            x = jnp.concatenate(
                [contrib[:, 128 * (8 * q + u):128 * (8 * q + u + 1)]
                 for u in range(8)], axis=0)
            o_ref[128 * q:128 * (q + 1), :] = x.T

    return pl.pallas_call(
        body,
        grid=grid,
        in_specs=[pl.BlockSpec((H, PREP_W), lambda i: (0, i))],
        out_specs=pl.BlockSpec((PREP_J, 128), lambda i: (i, 0)),
        out_shape=jax.ShapeDtypeStruct((NB * PREP_J, 128), jnp.float32),
        compiler_params=pltpu.CompilerParams(
            fuse_transposed_lhs_in_matmul=True),
    )(tableT)


def _sc_gather_sum(label_flat, contrib, B, L):
    bpw = B // NW
    GP = 32
    n_groups = bpw // GP
    ROWS = GP * L
    CH = 128
    NCH = ROWS // CH
    REM = ROWS - NCH * CH
    UNROLL = 10

    mesh = plsc.VectorSubcoreMesh(core_axis_name="c", subcore_axis_name="s")

    @functools.partial(
        pl.kernel,
        out_type=jax.ShapeDtypeStruct((H * B,), jnp.float32),
        mesh=mesh,
        scratch_types=[
            pltpu.VMEM((bpw * L,), jnp.int32),
            pltpu.VMEM((ROWS, H), jnp.float32),
            pltpu.VMEM((ROWS, H), jnp.float32),
            pltpu.VMEM((H * GP,), jnp.float32),
            pltpu.SemaphoreType.DMA,
            pltpu.SemaphoreType.DMA,
        ],
        compiler_params=pltpu.CompilerParams(
            needs_layout_passes=False, use_tc_tiling_on_sc=False),
    )
    def k(label_hbm, table_hbm, out_hbm, lab_v, rows0, rows1, outb_v,
          sem0, sem1):
        wid = lax.axis_index("s") * NC + lax.axis_index("c")
        base = wid * bpw
        pltpu.sync_copy(label_hbm.at[pl.ds(base * L, bpw * L)], lab_v)

        def remap_group(g):
            gbase = g * ROWS

            def remap(i, carry):
                off = pl.multiple_of(gbase + i * H, 16)
                r = lab_v[pl.ds(off, H)]
                p = (jnp.bitwise_and(r, -1024)
                     + jnp.left_shift(jnp.bitwise_and(r, 127), 3)
                     + jnp.bitwise_and(jnp.right_shift(r, 7), 7))
                lab_v[pl.ds(off, H)] = p
                return carry

            lax.fori_loop(0, ROWS // H, remap, 0)

        def chunks(g, rows_v, sem):
            goff = pl.multiple_of(g * ROWS, 8)
            cps = []
            for j in range(NCH):
                cps.append(pltpu.make_async_copy(
                    table_hbm.at[lab_v.at[pl.ds(goff + j * CH, CH)]],
                    rows_v.at[pl.ds(j * CH, CH)], sem))
            cps.append(pltpu.make_async_copy(
                table_hbm.at[lab_v.at[pl.ds(goff + NCH * CH, REM)]],
                rows_v.at[pl.ds(NCH * CH, REM)], sem))
            return cps

        def fire(g, rows_v, sem):
            for cp in chunks(g, rows_v, sem):
                cp.start()

        def drain(g, rows_v, sem):
            for cp in chunks(g, rows_v, sem):
                cp.wait()

        col_iota = lax.iota(jnp.int32, H) * GP

        def consume(g, rows_v):
            def per_b(b, carry):
                r0 = b * L
                seed = tuple(rows_v[r0 + u, :] for u in range(4))

                def accum(i, accs4):
                    r = r0 + 4 + i * 4
                    a0, a1, a2, a3 = accs4
                    return (a0 + rows_v[r, :], a1 + rows_v[r + 1, :],
                            a2 + rows_v[r + 2, :], a3 + rows_v[r + 3, :])

                n4 = (L - 4) // 4
                accs = list(lax.fori_loop(0, n4, accum, seed))
                for u in range((L - 4) % 4):
                    accs[u] = accs[u] + rows_v[r0 + 4 + n4 * 4 + u, :]
                acc = (accs[0] + accs[1]) + (accs[2] + accs[3])
                plsc.store_scatter(outb_v, [col_iota + b], acc)
                return carry
            lax.fori_loop(0, GP, per_b, 0)
            for c in range(H):
                pltpu.sync_copy(
                    outb_v.at[pl.ds(c * GP, GP)],
                    out_hbm.at[pl.ds(c * B + base + g * GP, GP)])

        remap_group(0)
        remap_group(1)
        fire(0, rows0, sem0)

        def per_pair(i, carry):
            g0 = i * 2
            fire(g0 + 1, rows1, sem1)

            @pl.when(g0 + 2 < n_groups)
            def _():
                remap_group(g0 + 2)

            drain(g0, rows0, sem0)
            consume(g0, rows0)

            @pl.when(g0 + 2 < n_groups)
            def _():
                fire(g0 + 2, rows0, sem0)

            @pl.when(g0 + 3 < n_groups)
            def _():
                remap_group(g0 + 3)

            drain(g0 + 1, rows1, sem1)
            consume(g0 + 1, rows1)
            return carry

        lax.fori_loop(0, n_groups // 2, per_pair, 0)

    return k(label_flat, contrib)


def _tc_finalize(accT, fixedT, wm, weT, wf, bias):
    B = accT.shape[1]
    F = fixedT.shape[0]
    BLK = 4096
    grid = (B // BLK,)

    def body(acc_ref, fx_ref, wm_ref, weT_ref, wf_ref, b_ref, out_ref):
        embed = jnp.sqrt(acc_ref[...])
        fx = fx_ref[...]
        v = jnp.dot(wm_ref[...], fx, preferred_element_type=jnp.float32)
        v = v + weT_ref[...]
        s1 = jnp.dot(wf_ref[...], fx, preferred_element_type=jnp.float32)
        logit = jnp.sum(embed * v, axis=0, keepdims=True) + s1 + b_ref[...]
        out_ref[...] = jax.nn.sigmoid(logit)

    return pl.pallas_call(
        body,
        grid=grid,
        in_specs=[
            pl.BlockSpec((H, BLK), lambda i: (0, i)),
            pl.BlockSpec((F, BLK), lambda i: (0, i)),
            pl.BlockSpec((H, F), lambda i: (0, 0)),
            pl.BlockSpec((H, 1), lambda i: (0, 0)),
            pl.BlockSpec((1, F), lambda i: (0, 0)),
            pl.BlockSpec((1, 1), lambda i: (0, 0)),
        ],
        out_specs=pl.BlockSpec((1, BLK), lambda i: (0, i)),
        out_shape=jax.ShapeDtypeStruct((1, B), jnp.float32),
    )(accT, fixedT, wm, weT, wf, bias)


def kernel(label, fixed, emb_table, final_w, final_b):
    B, L = label.shape
    F = fixed.shape[1]
    packed = _tc_prep(emb_table.T)
    contrib = packed.reshape(packed.shape[0] * 8, H)
    acc = _sc_gather_sum(label.reshape(-1).astype(jnp.int32), contrib, B, L)
    accT = acc.reshape(H, B)
    wf = final_w[:, :F]
    weT = final_w[0, F:F + H].reshape(H, 1)
    wm = final_w[0, F + H:].reshape(H, F)
    out = _tc_finalize(accT, fixed.T, wm, weT, wf, final_b.reshape(1, 1))
    return out.reshape(B, 1)
